# baseline (speedup 1.0000x reference)
import functools
import jax
import jax.numpy as jnp
from jax import lax
from jax.experimental import pallas as pl
from jax.experimental.pallas import tpu as pltpu
from jax.experimental.pallas import tpu_sc as plsc

_B, _NQ, _V, _NY, _D = 16384, 200, 100000, 14, 16
_NC, _NS, _L = 2, 16, 16
_NW = _NC * _NS
_BPW = _B // _NW
_UH = 256
_NU = 2 * _NQ
_NI = _NU // 8

_mesh = plsc.VectorSubcoreMesh(core_axis_name="c", subcore_axis_name="s")


@functools.partial(
    pl.kernel,
    out_type=jax.ShapeDtypeStruct((_NQ * 2 * 128 * 8 * 128,), jnp.float32),
    mesh=_mesh,
    scratch_types=[
        pltpu.VMEM((8, 128), jnp.int32),
        pltpu.VMEM((8, 128), jnp.int32),
        pltpu.VMEM((_UH, _D), jnp.float32),
        pltpu.VMEM((_UH, _D), jnp.float32),
        pltpu.VMEM((4096,), jnp.float32),
        pltpu.VMEM((4096,), jnp.float32),
        pltpu.VMEM((4096,), jnp.float32),
        pltpu.VMEM((4096,), jnp.float32),
        pltpu.VMEM((_NQ, _D), jnp.float32),
        pltpu.VMEM((16, 16), jnp.float32),
        pltpu.VMEM((_BPW,), jnp.int32),
        pltpu.VMEM_SHARED((_V, _D), jnp.float32),
        pltpu.SemaphoreType.DMA,
        pltpu.SemaphoreType.DMA,
        pltpu.SemaphoreType.DMA,
        pltpu.SemaphoreType.DMA,
        pltpu.SemaphoreType.DMA,
        pltpu.SemaphoreType.DMA,
    ],
    compiler_params=pltpu.CompilerParams(
        needs_layout_passes=False, use_tc_tiling_on_sc=False),
)
def _sc_embed(answ_hbm, table_hbm, qt_hbm, yst_hbm, year_hbm, out_hbm,
              idx_a, idx_b, gb0, gb1, sb0, sb1, sb2, sb3, qt_v, yst_v,
              year_v, table_spm, sg0, sg1, ss0, ss1, ss2, ss3):
    sid = lax.axis_index("s")
    wid = sid * _NC + lax.axis_index("c")
    wb = pl.multiple_of(wid * _BPW, _BPW)
    arow0 = pl.multiple_of(wid * (4 * _NQ), 8)
    gbufs, sbufs = (gb0, gb1), (sb0, sb1, sb2, sb3)
    sgs, sss = (sg0, sg1), (ss0, ss1, ss2, ss3)

    @pl.when(sid == 0)
    def _():
        pltpu.sync_copy(table_hbm, table_spm)

    pltpu.sync_copy(qt_hbm, qt_v)
    pltpu.sync_copy(yst_hbm, yst_v)
    pltpu.sync_copy(year_hbm.at[pl.ds(wb, _BPW)], year_v)
    plsc.subcore_barrier()

    iota = lax.iota(jnp.int32, _L)
    colc = [jnp.full((_L,), d, jnp.int32) for d in range(_D)]
    ysrows = [yst_v[d] for d in range(_D)]
    gdn = lax.GatherDimensionNumbers(
        offset_dims=(), collapsed_slice_dims=(0,), start_index_map=(0,))

    def stage_group(g, idx_v):
        pltpu.sync_copy(
            answ_hbm.at[pl.ds(pl.multiple_of(arow0 + g * 8, 8), 8)], idx_v)

    def fire_gather(k, i):
        p = k % 2
        qpar = (k // 2) % 2
        rows = (qpar * 4 + 2 * (k % 2), qpar * 4 + 2 * (k % 2) + 1)
        idx_v = idx_a if (k // 4) == 0 else idx_b
        for jj, r in enumerate(rows):
            pltpu.async_copy(table_spm.at[idx_v.at[r]],
                             gbufs[p].at[pl.ds(jj * 128, 128)], sgs[p])

    def wait_gather(p):
        pltpu.make_async_copy(table_hbm.at[pl.ds(0, _UH)], gbufs[p],
                              sgs[p]).wait()

    def fire_scatter(i, k):
        p = k % 4
        q = 4 * i + (k // 2)
        h = k % 2
        for dt in range(2):
            off = pl.multiple_of(
                (((q * 2 + dt) * 128 + 4 * wid + 2 * h) * 8) * 128, 2048)
            pltpu.async_copy(sbufs[p].at[pl.ds(dt * 2048, 2048)],
                             out_hbm.at[pl.ds(off, 2048)], sss[p])

    def wait_scatter(p):
        pltpu.make_async_copy(sbufs[p],
                              out_hbm.at[pl.ds(0, 4096)], sss[p]).wait()

    def transpose_bias(i, k):
        q = 4 * i + (k // 2)
        h = k % 2
        gbuf, sbuf = gbufs[k % 2], sbufs[k % 4]
        qrow = qt_v[q]
        csrows = [ysrows[d] + qrow[d] for d in range(_D)]
        for bt2 in range(2):
            boff = h * _UH + bt2 * 128

            def k16_body(k16, _, bt2=bt2, boff=boff):
                yearvec = year_v[pl.ds(boff + k16 * 16, 16)]
                win = gbuf.at[pl.ds(bt2 * 128 + k16 * 16, 16)]
                datas = [plsc.load_gather(win, [iota, colc[din]])
                         for din in range(_D)]
                ybs = [lax.gather(
                    csrows[din], yearvec[:, None], gdn, (1,),
                    mode=lax.GatherScatterMode.PROMISE_IN_BOUNDS)
                    for din in range(_D)]
                vals = [datas[din] + ybs[din]
                        for din in range(_D)]
                for din in range(_D):
                    soff = (((din // 8) * 2 + bt2) * 8 + (din % 8)) * 128
                    sbuf[pl.ds(soff + k16 * 16, 16)] = vals[din]
                return 0

            lax.fori_loop(0, 8, k16_body, 0)

    stage_group(0, idx_a)
    fire_gather(0, 0)

    def body(i, carry):
        for k in range(8):
            if k == 0:
                stage_group(2 * i + 1, idx_b)
            if k == 4:
                @pl.when(i < _NI - 1)
                def _():
                    stage_group(2 * i + 2, idx_a)
            if k < 7:
                fire_gather(k + 1, i)
            else:
                @pl.when(i < _NI - 1)
                def _():
                    fire_gather_next(i)
            wait_gather(k % 2)
            if k < 4:
                @pl.when(i > 0)
                def _():
                    wait_scatter(k % 4)
            else:
                wait_scatter(k % 4)
            transpose_bias(i, k)
            fire_scatter(i, k)
        return carry

    def fire_gather_next(i):
        rows = (0, 1)
        for jj, r in enumerate(rows):
            pltpu.async_copy(table_spm.at[idx_a.at[r]],
                             gbufs[0].at[pl.ds(jj * 128, 128)], sgs[0])

    lax.fori_loop(0, _NI, body, 0)
    wait_scatter(0)
    wait_scatter(1)
    wait_scatter(2)
    wait_scatter(3)


def kernel(year, answer, answer_table, yearly_table, question_table,
           alpha, beta):
    qt = beta[0] * question_table
    yst = jnp.zeros((16, 16), jnp.float32).at[:, :_NY].set(
        (alpha[0] * yearly_table).T)
    answ = (answer.astype(jnp.int32)
            .reshape(_NW, 4, 128, _NQ)
            .transpose(0, 3, 1, 2)
            .reshape(_NW * _NQ * 4, 128))
    year = year.astype(jnp.int32)
    out = _sc_embed(answ, answer_table, qt, yst, year)
    out5 = out.reshape(_NQ, 2, 128, 8, 128)
    return out5.transpose(2, 4, 0, 1, 3).reshape(_B, _NQ, _D)


# speedup vs baseline: 1.1645x; 1.1645x over previous
import functools
import jax
import jax.numpy as jnp
from jax import lax
from jax.experimental import pallas as pl
from jax.experimental.pallas import tpu as pltpu
from jax.experimental.pallas import tpu_sc as plsc

_B, _NQ, _V, _NY, _D = 16384, 200, 100000, 14, 16
_NC, _NS, _L = 2, 16, 16
_NW = _NC * _NS
_BPW = _B // _NW
_UH = 256
_NU = 2 * _NQ
_NI = _NU // 8

_mesh = plsc.VectorSubcoreMesh(core_axis_name="c", subcore_axis_name="s")


@functools.partial(
    pl.kernel,
    out_type=jax.ShapeDtypeStruct((_NQ * 2 * 128 * 8 * 128,), jnp.float32),
    mesh=_mesh,
    scratch_types=[
        pltpu.VMEM((8, 128), jnp.int32),
        pltpu.VMEM((8, 128), jnp.int32),
        pltpu.VMEM((_UH, _D), jnp.float32),
        pltpu.VMEM((_UH, _D), jnp.float32),
        pltpu.VMEM((_UH, _D), jnp.float32),
        pltpu.VMEM((_UH, _D), jnp.float32),
        pltpu.VMEM((4096,), jnp.float32),
        pltpu.VMEM((4096,), jnp.float32),
        pltpu.VMEM((_NQ, _D), jnp.float32),
        pltpu.VMEM((16, 16), jnp.float32),
        pltpu.VMEM((_BPW,), jnp.int32),
        pltpu.VMEM_SHARED((_V, _D), jnp.float32),
        pltpu.SemaphoreType.DMA,
        pltpu.SemaphoreType.DMA,
        pltpu.SemaphoreType.DMA,
        pltpu.SemaphoreType.DMA,
        pltpu.SemaphoreType.DMA,
        pltpu.SemaphoreType.DMA,
        pltpu.SemaphoreType.DMA,
        pltpu.SemaphoreType.DMA,
    ],
    compiler_params=pltpu.CompilerParams(
        needs_layout_passes=False, use_tc_tiling_on_sc=False),
)
def _sc_embed(answ_hbm, table_hbm, qt_hbm, yst_hbm, year_hbm, out_hbm,
              idx_a, idx_b, gb0, gb1, gb2, gb3, sb0, sb1, qt_v, yst_v,
              year_v, table_spm, sg0, sg1, sg2, sg3, ss0, ss1, si_a, si_b):
    sid = lax.axis_index("s")
    wid = sid * _NC + lax.axis_index("c")
    wb = pl.multiple_of(wid * _BPW, _BPW)
    arow0 = pl.multiple_of(wid * (4 * _NQ), 8)
    gbufs, sbufs = (gb0, gb1, gb2, gb3), (sb0, sb1)
    sgs, sss = (sg0, sg1, sg2, sg3), (ss0, ss1)

    @pl.when(sid == 0)
    def _():
        pltpu.sync_copy(table_hbm, table_spm)

    pltpu.sync_copy(qt_hbm, qt_v)
    pltpu.sync_copy(yst_hbm, yst_v)
    pltpu.sync_copy(year_hbm.at[pl.ds(wb, _BPW)], year_v)
    plsc.subcore_barrier()

    iota = lax.iota(jnp.int32, _L)
    colc = [jnp.full((_L,), d, jnp.int32) for d in range(_D)]
    ysrows = [yst_v[d] for d in range(_D)]
    gdn = lax.GatherDimensionNumbers(
        offset_dims=(), collapsed_slice_dims=(0,), start_index_map=(0,))

    def stage_group(g, idx_v, sem):
        pltpu.async_copy(
            answ_hbm.at[pl.ds(pl.multiple_of(arow0 + g * 8, 8), 8)], idx_v,
            sem)

    def wait_stage(idx_v, sem):
        pltpu.make_async_copy(answ_hbm.at[pl.ds(0, 8)], idx_v, sem).wait()

    def fire_gather(j, i):
        p = j % 4
        qpar = (j // 2) % 2
        rows = (qpar * 4 + 2 * (j % 2), qpar * 4 + 2 * (j % 2) + 1)
        idx_v = idx_a if (j // 4) % 2 == 0 else idx_b
        for jj, r in enumerate(rows):
            pltpu.async_copy(table_spm.at[idx_v.at[r]],
                             gbufs[p].at[pl.ds(jj * 128, 128)], sgs[p])

    def wait_gather(p):
        pltpu.make_async_copy(table_hbm.at[pl.ds(0, _UH)], gbufs[p],
                              sgs[p]).wait()

    def fire_scatter(i, k):
        p = k % 2
        q = 4 * i + (k // 2)
        h = k % 2
        for dt in range(2):
            off = pl.multiple_of(
                (((q * 2 + dt) * 128 + 4 * wid + 2 * h) * 8) * 128, 2048)
            pltpu.async_copy(sbufs[p].at[pl.ds(dt * 2048, 2048)],
                             out_hbm.at[pl.ds(off, 2048)], sss[p])

    def wait_scatter(p):
        pltpu.make_async_copy(sbufs[p],
                              out_hbm.at[pl.ds(0, 4096)], sss[p]).wait()

    def transpose_bias(i, k):
        q = 4 * i + (k // 2)
        h = k % 2
        gbuf, sbuf = gbufs[k % 4], sbufs[k % 2]
        qrow = qt_v[q]
        csrows = [ysrows[d] + qrow[d] for d in range(_D)]
        for bt2 in range(2):
            boff = h * _UH + bt2 * 128

            def k16_body(k16, _, bt2=bt2, boff=boff):
                yearvec = year_v[pl.ds(boff + k16 * 16, 16)]
                win = gbuf.at[pl.ds(bt2 * 128 + k16 * 16, 16)]
                datas = [plsc.load_gather(win, [iota, colc[din]])
                         for din in range(_D)]
                ybs = [lax.gather(
                    csrows[din], yearvec[:, None], gdn, (1,),
                    mode=lax.GatherScatterMode.PROMISE_IN_BOUNDS)
                    for din in range(_D)]
                vals = [datas[din] + ybs[din]
                        for din in range(_D)]
                for din in range(_D):
                    soff = (((din // 8) * 2 + bt2) * 8 + (din % 8)) * 128
                    sbuf[pl.ds(soff + k16 * 16, 16)] = vals[din]
                return 0

            lax.fori_loop(0, 8, k16_body, 0)

    stage_group(0, idx_a, si_a)
    wait_stage(idx_a, si_a)
    fire_gather(0, 0)
    fire_gather(1, 0)

    def body(i, carry):
        for k in range(8):
            if k == 0:
                stage_group(2 * i + 1, idx_b, si_b)
            if k == 1:
                wait_stage(idx_b, si_b)
            if k == 4:
                @pl.when(i < _NI - 1)
                def _():
                    stage_group(2 * i + 2, idx_a, si_a)
            if k == 5:
                @pl.when(i < _NI - 1)
                def _():
                    wait_stage(idx_a, si_a)
            if k < 6:
                fire_gather(k + 2, i)
            else:
                @pl.when(i < _NI - 1)
                def _():
                    fire_gather(k + 2, i)
            wait_gather(k % 4)
            if k < 2:
                @pl.when(i > 0)
                def _():
                    wait_scatter(k % 2)
            else:
                wait_scatter(k % 2)
            transpose_bias(i, k)
            fire_scatter(i, k)
        return carry

    lax.fori_loop(0, _NI, body, 0)
    wait_scatter(0)
    wait_scatter(1)


def kernel(year, answer, answer_table, yearly_table, question_table,
           alpha, beta):
    qt = beta[0] * question_table
    yst = jnp.zeros((16, 16), jnp.float32).at[:, :_NY].set(
        (alpha[0] * yearly_table).T)
    answ = (answer.astype(jnp.int32)
            .reshape(_NW, 4, 128, _NQ)
            .transpose(0, 3, 1, 2)
            .reshape(_NW * _NQ * 4, 128))
    year = year.astype(jnp.int32)
    out = _sc_embed(answ, answer_table, qt, yst, year)
    out5 = out.reshape(_NQ, 2, 128, 8, 128)
    return out5.transpose(2, 4, 0, 1, 3).reshape(_B, _NQ, _D)


# speedup vs baseline: 1.1881x; 1.0203x over previous
import functools
import jax
import jax.numpy as jnp
from jax import lax
from jax.experimental import pallas as pl
from jax.experimental.pallas import tpu as pltpu
from jax.experimental.pallas import tpu_sc as plsc

_B, _NQ, _V, _NY, _D = 16384, 200, 100000, 14, 16
_NC, _NS, _L = 2, 16, 16
_NW = _NC * _NS
_BPW = _B // _NW
_UH = 256
_NU = 2 * _NQ
_NI = _NU // 8

_mesh = plsc.VectorSubcoreMesh(core_axis_name="c", subcore_axis_name="s")


@functools.partial(
    pl.kernel,
    out_type=jax.ShapeDtypeStruct((_NQ * 2 * 128 * 8 * 128,), jnp.float32),
    mesh=_mesh,
    scratch_types=[
        pltpu.VMEM((8, 128), jnp.int32),
        pltpu.VMEM((8, 128), jnp.int32),
        pltpu.VMEM((_UH, _D), jnp.float32),
        pltpu.VMEM((_UH, _D), jnp.float32),
        pltpu.VMEM((4096,), jnp.float32),
        pltpu.VMEM((4096,), jnp.float32),
        pltpu.VMEM((4096,), jnp.float32),
        pltpu.VMEM((4096,), jnp.float32),
        pltpu.VMEM((_NQ, _D), jnp.float32),
        pltpu.VMEM((16, 16), jnp.float32),
        pltpu.VMEM((_BPW,), jnp.int32),
        pltpu.VMEM_SHARED((_V, _D), jnp.float32),
        pltpu.SemaphoreType.DMA,
        pltpu.SemaphoreType.DMA,
        pltpu.SemaphoreType.DMA,
        pltpu.SemaphoreType.DMA,
        pltpu.SemaphoreType.DMA,
        pltpu.SemaphoreType.DMA,
        pltpu.SemaphoreType.DMA,
        pltpu.SemaphoreType.DMA,
    ],
    compiler_params=pltpu.CompilerParams(
        needs_layout_passes=False, use_tc_tiling_on_sc=False),
)
def _sc_embed(answ_hbm, table_hbm, qt_hbm, yst_hbm, year_hbm, out_hbm,
              idx_a, idx_b, gb0, gb1, sb0, sb1, sb2, sb3, qt_v, yst_v,
              year_v, table_spm, sg0, sg1, ss0, ss1, ss2, ss3, si_a, si_b):
    sid = lax.axis_index("s")
    wid = sid * _NC + lax.axis_index("c")
    wb = pl.multiple_of(wid * _BPW, _BPW)
    arow0 = pl.multiple_of(wid * (4 * _NQ), 8)
    gbufs, sbufs = (gb0, gb1), (sb0, sb1, sb2, sb3)
    sgs, sss = (sg0, sg1), (ss0, ss1, ss2, ss3)

    @pl.when(sid == 0)
    def _():
        pltpu.sync_copy(table_hbm, table_spm)

    pltpu.sync_copy(qt_hbm, qt_v)
    pltpu.sync_copy(yst_hbm, yst_v)
    pltpu.sync_copy(year_hbm.at[pl.ds(wb, _BPW)], year_v)
    plsc.subcore_barrier()

    iota = lax.iota(jnp.int32, _L)
    colc = [jnp.full((_L,), d, jnp.int32) for d in range(_D)]
    ysrows = [yst_v[d] for d in range(_D)]
    gdn = lax.GatherDimensionNumbers(
        offset_dims=(), collapsed_slice_dims=(0,), start_index_map=(0,))

    def stage_group(g, idx_v, sem):
        pltpu.async_copy(
            answ_hbm.at[pl.ds(pl.multiple_of(arow0 + g * 8, 8), 8)], idx_v,
            sem)

    def wait_stage(idx_v, sem):
        pltpu.make_async_copy(answ_hbm.at[pl.ds(0, 8)], idx_v, sem).wait()

    def fire_gather(k, i):
        p = k % 2
        qpar = (k // 2) % 2
        rows = (qpar * 4 + 2 * (k % 2), qpar * 4 + 2 * (k % 2) + 1)
        idx_v = idx_a if (k // 4) == 0 else idx_b
        for jj, r in enumerate(rows):
            pltpu.async_copy(table_spm.at[idx_v.at[r]],
                             gbufs[p].at[pl.ds(jj * 128, 128)], sgs[p])

    def wait_gather(p):
        pltpu.make_async_copy(table_hbm.at[pl.ds(0, _UH)], gbufs[p],
                              sgs[p]).wait()

    def fire_scatter(i, k):
        p = k % 4
        q = 4 * i + (k // 2)
        h = k % 2
        for dt in range(2):
            off = pl.multiple_of(
                (((q * 2 + dt) * 128 + 4 * wid + 2 * h) * 8) * 128, 2048)
            pltpu.async_copy(sbufs[p].at[pl.ds(dt * 2048, 2048)],
                             out_hbm.at[pl.ds(off, 2048)], sss[p])

    def wait_scatter(p):
        pltpu.make_async_copy(sbufs[p],
                              out_hbm.at[pl.ds(0, 4096)], sss[p]).wait()

    def transpose_bias(i, k):
        q = 4 * i + (k // 2)
        h = k % 2
        gbuf, sbuf = gbufs[k % 2], sbufs[k % 4]
        qrow = qt_v[q]
        csrows = [ysrows[d] + qrow[d] for d in range(_D)]
        for bt2 in range(2):
            boff = h * _UH + bt2 * 128

            def k16_body(k16, _, bt2=bt2, boff=boff):
                yearvec = year_v[pl.ds(boff + k16 * 16, 16)]
                win = gbuf.at[pl.ds(bt2 * 128 + k16 * 16, 16)]
                datas = [plsc.load_gather(win, [iota, colc[din]])
                         for din in range(_D)]
                ybs = [lax.gather(
                    csrows[din], yearvec[:, None], gdn, (1,),
                    mode=lax.GatherScatterMode.PROMISE_IN_BOUNDS)
                    for din in range(_D)]
                vals = [datas[din] + ybs[din]
                        for din in range(_D)]
                for din in range(_D):
                    soff = (((din // 8) * 2 + bt2) * 8 + (din % 8)) * 128
                    sbuf[pl.ds(soff + k16 * 16, 16)] = vals[din]
                return 0

            lax.fori_loop(0, 8, k16_body, 0)

    stage_group(0, idx_a, si_a)
    wait_stage(idx_a, si_a)
    fire_gather(0, 0)

    def body(i, carry):
        for k in range(8):
            if k == 0:
                stage_group(2 * i + 1, idx_b, si_b)
            if k == 4:
                @pl.when(i < _NI - 1)
                def _():
                    stage_group(2 * i + 2, idx_a, si_a)
            if k == 3:
                wait_stage(idx_b, si_b)
            if k < 7:
                fire_gather(k + 1, i)
            else:
                @pl.when(i < _NI - 1)
                def _():
                    wait_stage(idx_a, si_a)
                    fire_gather_next(i)
            wait_gather(k % 2)
            if k < 4:
                @pl.when(i > 0)
                def _():
                    wait_scatter(k % 4)
            else:
                wait_scatter(k % 4)
            transpose_bias(i, k)
            fire_scatter(i, k)
        return carry

    def fire_gather_next(i):
        rows = (0, 1)
        for jj, r in enumerate(rows):
            pltpu.async_copy(table_spm.at[idx_a.at[r]],
                             gbufs[0].at[pl.ds(jj * 128, 128)], sgs[0])

    lax.fori_loop(0, _NI, body, 0)
    wait_scatter(0)
    wait_scatter(1)
    wait_scatter(2)
    wait_scatter(3)


def kernel(year, answer, answer_table, yearly_table, question_table,
           alpha, beta):
    qt = beta[0] * question_table
    yst = jnp.zeros((16, 16), jnp.float32).at[:, :_NY].set(
        (alpha[0] * yearly_table).T)
    answ = (answer.astype(jnp.int32)
            .reshape(_NW, 4, 128, _NQ)
            .transpose(0, 3, 1, 2)
            .reshape(_NW * _NQ * 4, 128))
    year = year.astype(jnp.int32)
    out = _sc_embed(answ, answer_table, qt, yst, year)
    out5 = out.reshape(_NQ, 2, 128, 8, 128)
    return out5.transpose(2, 4, 0, 1, 3).reshape(_B, _NQ, _D)
